# baseline (speedup 1.0000x reference)
import functools

import jax
import jax.numpy as jnp
from jax import lax
from jax.experimental import pallas as pl
from jax.experimental.pallas import tpu as pltpu
from jax.experimental.pallas import tpu_sc as plsc

_K = 128
_HN = 512
_BMM = 1024
_BM = 64
_BISECT_ITERS = 20


def _matmul_body(x_ref, w_ref, b_ref, h_ref):
    i = pl.program_id(1)
    bmm = h_ref.shape[0]
    xb = x_ref[pl.ds(i * bmm, bmm), :]
    z = jax.lax.dot_general(
        xb, w_ref[...], (((1,), (1,)), ((), ())),
        preferred_element_type=jnp.float32,
        precision=jax.lax.Precision.DEFAULT,
    )
    z = z + b_ref[...]
    h_ref[...] = 1.0 / (1.0 + jnp.exp(-z))


_NC = 2
_NS = 16
_NBUCKET = 1024
_CAND_CAP = 1024


def _sc_select_body(h_hbm, thr_hbm, row_v, hist_v, cand_v, thr_v):
    nw = _NC * _NS
    b, dh = h_hbm.shape
    rows_per = b // nw
    ngrp = dh // 16
    wid = lax.axis_index("s") * _NC + lax.axis_index("c")
    kf = jnp.float32(_K)
    zeros16 = jnp.zeros((16,), jnp.float32)
    ones16 = jnp.ones((16,), jnp.float32)
    iota16 = lax.iota(jnp.int32, 16)
    iota16f = iota16.astype(jnp.float32)

    def row_body(r, thr_stage):
        row = wid * rows_per + r
        pltpu.sync_copy(h_hbm.at[row], row_v)

        def z_body(g, _):
            hist_v[pl.ds(g * 16, 16)] = zeros16
            return 0
        lax.fori_loop(0, _NBUCKET // 16, z_body, 0)

        def h_body(j, _):
            v = row_v[pl.ds(j * 16, 16)]
            bk = lax.shift_right_logical(plsc.bitcast(v, jnp.int32), 20)
            plsc.addupdate_scatter(hist_v, [bk], ones16)
            return 0
        lax.fori_loop(0, ngrp, h_body, 0)

        def s_body(g, carry):
            cum, bstar_acc, cabove_acc = carry
            gi = _NBUCKET // 16 - 1 - g
            grp = hist_v[pl.ds(gi * 16, 16)]
            rev = lax.rev(grp, (0,))
            cs = plsc.cumsum(rev) + cum
            csprev = cs - rev
            is_k = jnp.logical_and(cs >= kf, csprev < kf)
            ids_desc = jnp.float32(gi * 16 + 15) - iota16f
            bstar_acc = bstar_acc + jnp.where(is_k, ids_desc, 0.0)
            cabove_acc = cabove_acc + jnp.where(is_k, csprev, 0.0)
            return cum + jnp.sum(grp), bstar_acc, cabove_acc
        _, bstar_v16, cabove_v16 = lax.fori_loop(
            0, _NBUCKET // 16, s_body, (jnp.float32(0.0), zeros16, zeros16))
        bstar = jnp.sum(bstar_v16).astype(jnp.int32)
        cabove = jnp.sum(cabove_v16)

        def c_body(j, off):
            v = row_v[pl.ds(j * 16, 16)]
            bits = plsc.bitcast(v, jnp.int32)
            m = lax.shift_right_logical(bits, 20) == bstar
            off_use = jnp.minimum(off, _CAND_CAP - 16)
            plsc.store_compressed(cand_v.at[pl.ds(off_use, 16)], bits, mask=m)
            n = jnp.max(plsc.all_reduce_population_count(m))
            return off + n
        noff = lax.fori_loop(0, ngrp, c_body, jnp.int32(0))
        noff = jnp.minimum(noff, _CAND_CAP)
        nv = (noff + 15) // 16

        def bit_body(t, prefix):
            cand_t = prefix | lax.shift_left(jnp.int32(1), 19 - t)

            def cnt_body(u, acc):
                cb = cand_v[pl.ds(u * 16, 16)]
                valid = (u * 16 + iota16) < noff
                hit = jnp.logical_and(valid, cb >= cand_t)
                return acc + jnp.sum(jnp.where(hit, 1.0, 0.0))
            cnt = lax.fori_loop(0, nv, cnt_body, cabove)
            return jnp.where(cnt >= kf, cand_t, prefix)
        thr_bits = lax.fori_loop(
            0, 20, bit_body, lax.shift_left(bstar, 20))
        thr = plsc.bitcast(jnp.broadcast_to(thr_bits, (16,)), jnp.float32)

        lane = lax.rem(r, 16)
        thr_stage = jnp.where(iota16 == lane, thr, thr_stage)

        @pl.when(lane == 15)
        def _():
            thr_v[pl.ds((r // 16) * 16, 16)] = thr_stage
        return thr_stage

    lax.fori_loop(0, rows_per, row_body, zeros16)
    pltpu.sync_copy(thr_v, thr_hbm.at[pl.ds(wid * rows_per, rows_per)])


def _sc_select(hidden):
    b, dh = hidden.shape
    rows_per = b // (_NC * _NS)
    return pl.kernel(
        _sc_select_body,
        out_type=jax.ShapeDtypeStruct((b,), jnp.float32),
        mesh=plsc.VectorSubcoreMesh(
            core_axis_name="c", subcore_axis_name="s",
            num_cores=_NC, num_subcores=_NS),
        scratch_types=[
            pltpu.VMEM((dh,), jnp.float32),
            pltpu.VMEM((_NBUCKET,), jnp.float32),
            pltpu.VMEM((_CAND_CAP,), jnp.int32),
            pltpu.VMEM((rows_per,), jnp.float32),
        ],
        compiler_params=pltpu.CompilerParams(needs_layout_passes=False),
    )(hidden)


def _mask_body(h_ref, t_ref, o_ref):
    h = h_ref[...]
    o_ref[...] = jnp.where(h >= t_ref[...], h, 0.0)


def _select_body(h_ref, o_ref):
    h = h_ref[...]
    bm, dh = h.shape
    kf = jnp.float32(_K)

    def it(_, lohi):
        lo, hi = lohi
        mid = 0.5 * (lo + hi)
        cnt = jnp.sum((h >= mid).astype(jnp.float32), axis=1, keepdims=True)
        big = cnt >= kf
        return jnp.where(big, mid, lo), jnp.where(big, hi, mid)

    m = h
    while m.shape[1] > _K:
        s = m.shape[1] // 2
        m = jnp.maximum(m[:, :s], m[:, s:])
    t1 = jnp.min(m, axis=1, keepdims=True)
    gm = jnp.max(m, axis=1, keepdims=True) + jnp.float32(2.0 ** -18)

    lo, _ = jax.lax.fori_loop(0, _BISECT_ITERS, it, (t1, gm))
    o_ref[...] = jnp.where(h >= lo, h, 0.0)


def kernel(x, W, b):
    B, DIN = x.shape
    DH = W.shape[0]
    hn = min(_HN, DH)
    bmm = min(_BMM, B)
    bm = min(_BM, B)
    b2 = b.reshape(1, DH)

    hidden = pl.pallas_call(
        _matmul_body,
        grid=(DH // hn, B // bmm),
        in_specs=[
            pl.BlockSpec((B, DIN), lambda j, i: (0, 0)),
            pl.BlockSpec((hn, DIN), lambda j, i: (j, 0)),
            pl.BlockSpec((1, hn), lambda j, i: (0, j)),
        ],
        out_specs=pl.BlockSpec((bmm, hn), lambda j, i: (i, j)),
        out_shape=jax.ShapeDtypeStruct((B, DH), jnp.float32),
        compiler_params=pltpu.CompilerParams(
            dimension_semantics=("arbitrary", "arbitrary"),
            vmem_limit_bytes=60 * 1024 * 1024,
        ),
    )(x, W, b2)

    thr = _sc_select(hidden)

    out = pl.pallas_call(
        _mask_body,
        grid=(B // bm,),
        in_specs=[
            pl.BlockSpec((bm, DH), lambda i: (i, 0)),
            pl.BlockSpec((bm, 1), lambda i: (i, 0)),
        ],
        out_specs=pl.BlockSpec((bm, DH), lambda i: (i, 0)),
        out_shape=jax.ShapeDtypeStruct((B, DH), jnp.float32),
        input_output_aliases={0: 0},
        compiler_params=pltpu.CompilerParams(
            dimension_semantics=("parallel",),
            vmem_limit_bytes=60 * 1024 * 1024,
        ),
    )(hidden, thr.reshape(B, 1))
    return out


# speedup vs baseline: 3.4386x; 3.4386x over previous
import jax
import jax.numpy as jnp
from jax.experimental import pallas as pl
from jax.experimental.pallas import tpu as pltpu

_K = 128
_HN = 512
_BMM = 1024
_BM = 64
_BISECT_ITERS = 18


def _matmul_body(x_ref, w_ref, b_ref, h_ref, tm_ref):
    i = pl.program_id(1)
    bmm = h_ref.shape[0]
    xb = x_ref[pl.ds(i * bmm, bmm), :]
    z = jax.lax.dot_general(
        xb, w_ref[...], (((1,), (1,)), ((), ())),
        preferred_element_type=jnp.float32,
        precision=jax.lax.Precision.DEFAULT,
    )
    z = z + b_ref[...]
    h = 1.0 / (1.0 + jnp.exp(-z))
    h_ref[...] = h
    j = pl.program_id(0)
    m = h
    while m.shape[1] > 128:
        s = m.shape[1] // 2
        m = jnp.maximum(m[:, :s], m[:, s:])
    row0 = pl.multiple_of(i * bmm, 8)
    prev = tm_ref[pl.ds(row0, bmm), :]
    tm_ref[pl.ds(row0, bmm), :] = jnp.where(
        j == 0, m, jnp.maximum(prev, m))


def _select_body(h_ref, tm_ref, o_ref):
    h = h_ref[...]
    kf = jnp.float32(_K)

    def it(_, lohi):
        lo, hi = lohi
        mid = 0.5 * (lo + hi)
        cnt = jnp.sum((h >= mid).astype(jnp.float32), axis=1, keepdims=True)
        big = cnt >= kf
        return jnp.where(big, mid, lo), jnp.where(big, hi, mid)

    tm = tm_ref[...]
    if tm.shape[1] >= _K:
        t1 = jnp.min(tm, axis=1, keepdims=True)
    else:
        t1 = jnp.zeros((tm.shape[0], 1), jnp.float32)
    gm = jnp.max(tm, axis=1, keepdims=True) + jnp.float32(2.0 ** -18)

    lo, _ = jax.lax.fori_loop(0, _BISECT_ITERS, it, (t1, gm))
    o_ref[...] = jnp.where(h >= lo, h, 0.0)


def kernel(x, W, b):
    B, DIN = x.shape
    DH = W.shape[0]
    hn = min(_HN, DH)
    bmm = min(_BMM, B)
    bm = min(_BM, B)
    b2 = b.reshape(1, DH)

    nh = DH // hn
    hidden, tmax = pl.pallas_call(
        _matmul_body,
        grid=(nh, B // bmm),
        in_specs=[
            pl.BlockSpec((B, DIN), lambda j, i: (0, 0)),
            pl.BlockSpec((hn, DIN), lambda j, i: (j, 0)),
            pl.BlockSpec((1, hn), lambda j, i: (0, j)),
        ],
        out_specs=[
            pl.BlockSpec((bmm, hn), lambda j, i: (i, j)),
            pl.BlockSpec((B, 128), lambda j, i: (0, 0)),
        ],
        out_shape=[
            jax.ShapeDtypeStruct((B, DH), jnp.float32),
            jax.ShapeDtypeStruct((B, 128), jnp.float32),
        ],
        compiler_params=pltpu.CompilerParams(
            dimension_semantics=("arbitrary", "arbitrary"),
            vmem_limit_bytes=60 * 1024 * 1024,
        ),
    )(x, W, b2)

    out = pl.pallas_call(
        _select_body,
        grid=(B // bm,),
        in_specs=[
            pl.BlockSpec((bm, DH), lambda i: (i, 0)),
            pl.BlockSpec((bm, 128), lambda i: (i, 0)),
        ],
        out_specs=pl.BlockSpec((bm, DH), lambda i: (i, 0)),
        out_shape=jax.ShapeDtypeStruct((B, DH), jnp.float32),
        input_output_aliases={0: 0},
        compiler_params=pltpu.CompilerParams(
            dimension_semantics=("parallel",),
            vmem_limit_bytes=60 * 1024 * 1024,
        ),
    )(hidden, tmax)
    return out


# speedup vs baseline: 3.4993x; 1.0177x over previous
import jax
import jax.numpy as jnp
from jax.experimental import pallas as pl
from jax.experimental.pallas import tpu as pltpu

_K = 128
_HN = 512
_BMM = 1024
_BM = 64
_BISECT_ITERS = 18


def _matmul_body(x_ref, w_ref, b_ref, h_ref, tm_ref):
    i = pl.program_id(1)
    bmm = h_ref.shape[0]
    xb = x_ref[pl.ds(i * bmm, bmm), :]
    z = jax.lax.dot_general(
        xb, w_ref[...], (((1,), (1,)), ((), ())),
        preferred_element_type=jnp.float32,
        precision=jax.lax.Precision.DEFAULT,
    )
    z = z + b_ref[...]
    h = 1.0 / (1.0 + jnp.exp(-z))
    h_ref[...] = h
    j = pl.program_id(0)
    m = h
    while m.shape[1] > 128:
        s = m.shape[1] // 2
        m = jnp.maximum(m[:, :s], m[:, s:])
    row0 = pl.multiple_of(i * bmm, 8)
    prev = tm_ref[pl.ds(row0, bmm), :]
    tm_ref[pl.ds(row0, bmm), :] = jnp.where(
        j == 0, m, jnp.maximum(prev, m))


def _select_body(h_ref, tm_ref, o_ref):
    kf = jnp.float32(_K)

    def it(_, lohi):
        lo, hi = lohi
        mid = 0.5 * (lo + hi)
        cnt = jnp.sum((h_ref[...] >= mid).astype(jnp.float32), axis=1,
                      keepdims=True)
        big = cnt >= kf
        return jnp.where(big, mid, lo), jnp.where(big, hi, mid)

    tm = tm_ref[...]
    if tm.shape[1] >= _K:
        t1 = jnp.min(tm, axis=1, keepdims=True)
    else:
        t1 = jnp.zeros((tm.shape[0], 1), jnp.float32)
    gm = jnp.max(tm, axis=1, keepdims=True) + jnp.float32(2.0 ** -18)

    lo, _ = jax.lax.fori_loop(0, _BISECT_ITERS, it, (t1, gm))
    h = h_ref[...]
    o_ref[...] = jnp.where(h >= lo, h, 0.0)


def kernel(x, W, b):
    B, DIN = x.shape
    DH = W.shape[0]
    hn = min(_HN, DH)
    bmm = min(_BMM, B)
    bm = min(_BM, B)
    b2 = b.reshape(1, DH)

    nh = DH // hn
    hidden, tmax = pl.pallas_call(
        _matmul_body,
        grid=(nh, B // bmm),
        in_specs=[
            pl.BlockSpec((B, DIN), lambda j, i: (0, 0)),
            pl.BlockSpec((hn, DIN), lambda j, i: (j, 0)),
            pl.BlockSpec((1, hn), lambda j, i: (0, j)),
        ],
        out_specs=[
            pl.BlockSpec((bmm, hn), lambda j, i: (i, j)),
            pl.BlockSpec((B, 128), lambda j, i: (0, 0)),
        ],
        out_shape=[
            jax.ShapeDtypeStruct((B, DH), jnp.float32),
            jax.ShapeDtypeStruct((B, 128), jnp.float32),
        ],
        compiler_params=pltpu.CompilerParams(
            dimension_semantics=("arbitrary", "arbitrary"),
            vmem_limit_bytes=60 * 1024 * 1024,
        ),
    )(x, W, b2)

    out = pl.pallas_call(
        _select_body,
        grid=(B // bm,),
        in_specs=[
            pl.BlockSpec((bm, DH), lambda i: (i, 0)),
            pl.BlockSpec((bm, 128), lambda i: (i, 0)),
        ],
        out_specs=pl.BlockSpec((bm, DH), lambda i: (i, 0)),
        out_shape=jax.ShapeDtypeStruct((B, DH), jnp.float32),
        input_output_aliases={0: 0},
        compiler_params=pltpu.CompilerParams(
            dimension_semantics=("parallel",),
            vmem_limit_bytes=60 * 1024 * 1024,
        ),
    )(hidden, tmax)
    return out


# speedup vs baseline: 3.6707x; 1.0490x over previous
import jax
import jax.numpy as jnp
from jax.experimental import pallas as pl
from jax.experimental.pallas import tpu as pltpu

_K = 128
_HN = 1024
_BMM = 1024
_BM = 64
_BISECT_ITERS = 18


def _matmul_body(x_ref, w_ref, b_ref, h_ref, tm_ref):
    i = pl.program_id(1)
    bmm = h_ref.shape[0]
    xb = x_ref[pl.ds(i * bmm, bmm), :]
    z = jax.lax.dot_general(
        xb, w_ref[...], (((1,), (1,)), ((), ())),
        preferred_element_type=jnp.float32,
        precision=jax.lax.Precision.DEFAULT,
    )
    z = z + b_ref[...]
    h = 1.0 / (1.0 + jnp.exp(-z))
    h_ref[...] = h
    j = pl.program_id(0)
    m = h
    while m.shape[1] > 128:
        s = m.shape[1] // 2
        m = jnp.maximum(m[:, :s], m[:, s:])
    row0 = pl.multiple_of(i * bmm, 8)
    prev = tm_ref[pl.ds(row0, bmm), :]
    tm_ref[pl.ds(row0, bmm), :] = jnp.where(
        j == 0, m, jnp.maximum(prev, m))


def _select_body(h_ref, tm_ref, o_ref):
    kf = jnp.float32(_K)

    def it(_, lohi):
        lo, hi = lohi
        mid = 0.5 * (lo + hi)
        cnt = jnp.sum((h_ref[...] >= mid).astype(jnp.float32), axis=1,
                      keepdims=True)
        big = cnt >= kf
        return jnp.where(big, mid, lo), jnp.where(big, hi, mid)

    tm = tm_ref[...]
    if tm.shape[1] >= _K:
        t1 = jnp.min(tm, axis=1, keepdims=True)
    else:
        t1 = jnp.zeros((tm.shape[0], 1), jnp.float32)
    gm = jnp.max(tm, axis=1, keepdims=True) + jnp.float32(2.0 ** -18)

    lo, _ = jax.lax.fori_loop(0, _BISECT_ITERS, it, (t1, gm))
    h = h_ref[...]
    o_ref[...] = jnp.where(h >= lo, h, 0.0)


def kernel(x, W, b):
    B, DIN = x.shape
    DH = W.shape[0]
    hn = min(_HN, DH)
    bmm = min(_BMM, B)
    bm = min(_BM, B)
    b2 = b.reshape(1, DH)

    nh = DH // hn
    hidden, tmax = pl.pallas_call(
        _matmul_body,
        grid=(nh, B // bmm),
        in_specs=[
            pl.BlockSpec((B, DIN), lambda j, i: (0, 0)),
            pl.BlockSpec((hn, DIN), lambda j, i: (j, 0)),
            pl.BlockSpec((1, hn), lambda j, i: (0, j)),
        ],
        out_specs=[
            pl.BlockSpec((bmm, hn), lambda j, i: (i, j)),
            pl.BlockSpec((B, 128), lambda j, i: (0, 0)),
        ],
        out_shape=[
            jax.ShapeDtypeStruct((B, DH), jnp.float32),
            jax.ShapeDtypeStruct((B, 128), jnp.float32),
        ],
        compiler_params=pltpu.CompilerParams(
            dimension_semantics=("arbitrary", "arbitrary"),
            vmem_limit_bytes=60 * 1024 * 1024,
        ),
    )(x, W, b2)

    out = pl.pallas_call(
        _select_body,
        grid=(B // bm,),
        in_specs=[
            pl.BlockSpec((bm, DH), lambda i: (i, 0)),
            pl.BlockSpec((bm, 128), lambda i: (i, 0)),
        ],
        out_specs=pl.BlockSpec((bm, DH), lambda i: (i, 0)),
        out_shape=jax.ShapeDtypeStruct((B, DH), jnp.float32),
        input_output_aliases={0: 0},
        compiler_params=pltpu.CompilerParams(
            dimension_semantics=("parallel",),
            vmem_limit_bytes=60 * 1024 * 1024,
        ),
    )(hidden, tmax)
    return out
